# baseline (speedup 1.0000x reference)
import functools

import jax
import jax.numpy as jnp
from jax import lax
from jax.experimental import pallas as pl
from jax.experimental.pallas import tpu as pltpu
from jax.experimental.pallas import tpu_sc as plsc

VOCAB = 1000000
TCB = 512
HALF = TCB * 977
D = 64
NW = 32
BQ = 4096
LQ = 200
BBLK = 128
LCH = 2
CHUNK = BBLK * LCH
NCH = LQ // LCH
LANES = 16


def _tc_pack_kernel(w1_ref, w2_ref, o_ref):
    o_ref[:, 0:D] = w1_ref[...].T * 8.0
    o_ref[:, D:2 * D] = w2_ref[...].T * 8.0


def _tc_pack(wt):
    grid = HALF // TCB
    return pl.pallas_call(
        _tc_pack_kernel,
        grid=(grid,),
        in_specs=[
            pl.BlockSpec((D, TCB), lambda c: (0, c)),
            pl.BlockSpec((D, TCB), lambda c: (0, c + HALF // TCB)),
        ],
        out_specs=pl.BlockSpec((TCB, 2 * D), lambda c: (c, 0)),
        out_shape=jax.ShapeDtypeStruct((HALF, 2 * D), jnp.float32),
    )(wt, wt)


def _sc_gather_build():
    mesh = plsc.VectorSubcoreMesh(core_axis_name="c", subcore_axis_name="s")

    @functools.partial(
        pl.kernel,
        out_type=jax.ShapeDtypeStruct((LQ, D, BQ), jnp.float32),
        mesh=mesh,
        compiler_params=pltpu.CompilerParams(needs_layout_passes=False),
        scratch_types=[
            pltpu.VMEM((CHUNK,), jnp.int32),
            pltpu.VMEM((CHUNK,), jnp.int32),
            pltpu.VMEM((CHUNK,), jnp.int32),
            pltpu.VMEM((CHUNK,), jnp.int32),
            pltpu.VMEM((CHUNK, 2 * D), jnp.float32),
            pltpu.VMEM((CHUNK, 2 * D), jnp.float32),
            pltpu.VMEM((LCH, D, BBLK), jnp.float32),
            pltpu.SemaphoreType.DMA,
            pltpu.SemaphoreType.DMA,
            pltpu.SemaphoreType.DMA,
            pltpu.SemaphoreType.DMA,
        ],
    )
    def emb_kernel(xp_hbm, tab_hbm, out_hbm, idx0, idx1, kx0, kx1, gb0, gb1,
                   stag, si0, si1, sg0, sg1):
        wid = lax.axis_index("s") * 2 + lax.axis_index("c")
        idx_v = (idx0, idx1)
        kx = (kx0, kx1)
        gb = (gb0, gb1)
        si = (si0, si1)
        sg = (sg0, sg1)

        def start_fetch(g, p):
            pltpu.async_copy(xp_hbm.at[wid, g], idx_v[p], si[p])

        def wait_fetch(g, p):
            pltpu.make_async_copy(xp_hbm.at[wid, g], idx_v[p], si[p]).wait()

        def to_pair_rows(p):
            def tbody(q, c):
                sl = pl.ds(q * LANES, LANES)
                iv = idx_v[p][sl]
                kx[p][sl] = iv - jnp.where(iv >= HALF, HALF, 0)
                return c

            lax.fori_loop(0, CHUNK // LANES, tbody, 0)

        def start_gather(p):
            pltpu.async_copy(tab_hbm.at[kx[p]], gb[p], sg[p])

        def wait_gather(p):
            pltpu.make_async_copy(tab_hbm.at[kx[p]], gb[p], sg[p]).wait()

        iotas = [
            lax.iota(jnp.int32, LANES) + j * LANES
            for j in range(D // LANES)
        ]

        def stage_chunk(p):
            buf = gb[p]
            ix = idx_v[p]

            def grp_body(g16, c):
                r0 = g16 * LANES
                ivs = ix[pl.ds(r0, LANES)]
                offs = jnp.where(ivs >= HALF, D, 0).astype(jnp.int32)
                il = jnp.full((LANES,), r0 // BBLK, jnp.int32)
                rb0 = r0 % BBLK
                for k in range(LANES):
                    off = offs[k]
                    ib = jnp.full((LANES,), rb0 + k, jnp.int32)
                    for j in range(D // LANES):
                        v = buf[r0 + k, pl.ds(off + j * LANES, LANES)]
                        plsc.store_scatter(stag, [il, iotas[j], ib], v)
                return c

            lax.fori_loop(0, CHUNK // LANES, grp_body, 0)

        def wb_chunk(g):
            pltpu.sync_copy(
                stag, out_hbm.at[pl.ds(g * LCH, LCH), :,
                                 pl.ds(wid * BBLK, BBLK)])

        def step(g, p):
            g1 = jnp.minimum(g + 1, NCH - 1)
            g2 = jnp.minimum(g + 2, NCH - 1)
            wait_fetch(g1, 1 - p)
            to_pair_rows(1 - p)
            start_gather(1 - p)
            wait_gather(p)
            stage_chunk(p)
            start_fetch(g2, p)
            wb_chunk(g)

        start_fetch(0, 0)
        wait_fetch(0, 0)
        to_pair_rows(0)
        start_gather(0)
        start_fetch(1, 1)

        def group_body(t, c):
            step(2 * t, 0)
            step(2 * t + 1, 1)
            return c

        lax.fori_loop(0, NCH // 2, group_body, 0)
        wait_gather(0)
        wait_fetch(NCH - 1, 1)

    return emb_kernel


def kernel(x, embedding_weight):
    tab = _tc_pack(embedding_weight.T)
    xp = (x.astype(jnp.int32)
          .reshape(NW, BBLK, NCH, LCH)
          .transpose(0, 2, 3, 1)
          .reshape(NW, NCH, CHUNK))
    o = _sc_gather_build()(xp, tab)
    return jnp.transpose(o, (2, 0, 1))
